# baseline (speedup 1.0000x reference)
import functools

import jax
import jax.numpy as jnp
from jax import lax
from jax.experimental import pallas as pl
from jax.experimental.pallas import tpu as pltpu
from jax.experimental.pallas import tpu_sc as plsc

_BETA_CE = 10.0
_BETA_CU = 10.0
_MA_WINDOW = 100
_CAPACITY = 0.5


def _signal(d_st, d_ch, raw_o_ce, raw_m_cu):
    B, T = d_st.shape
    o_ce_pos = jax.nn.softplus(raw_o_ce)
    m_cu_pos = jax.nn.softplus(raw_m_cu)
    CE = d_st - (d_ch - jnp.log(o_ce_pos + 1e-10))
    W = min(_MA_WINDOW, T)
    if W <= 1:
        ma = d_st
    else:
        pad = jnp.repeat(d_st[:, :1], W - 1, axis=1)
        padded = jnp.concatenate([pad, d_st], axis=1)
        cs = jnp.cumsum(padded, axis=1)
        cs = jnp.concatenate([jnp.zeros((B, 1), dtype=d_st.dtype), cs], axis=1)
        ma = (cs[:, W:] - cs[:, :-W]) / W
    CU = d_st - m_cu_pos * ma
    S_CE = jax.nn.sigmoid(_BETA_CE * CE)
    S_CU = jax.nn.sigmoid(_BETA_CU * CU)
    return S_CE + S_CU - S_CE * S_CU


def _rank_body(g_ref, rank_ref):
    B, T = g_ref.shape
    CH = 256
    jj = lax.broadcasted_iota(jnp.int32, (CH, T), 1)
    for b in range(B):
        g_row = g_ref[b, :].reshape(1, T)
        for c in range(T // CH):
            vi = g_ref[b, c * CH:(c + 1) * CH].reshape(CH, 1)
            ii = lax.broadcasted_iota(jnp.int32, (CH, 1), 0) + (c * CH)
            before = (g_row > vi) | ((g_row == vi) & (jj < ii))
            cnt = jnp.sum(before.astype(jnp.float32), axis=1)
            rank_ref[b, c * CH:(c + 1) * CH] = cnt.astype(jnp.int32)


def _ranks(g):
    B, T = g.shape
    return pl.pallas_call(
        _rank_body,
        out_shape=jax.ShapeDtypeStruct((B, T), jnp.int32),
    )(g)


def _make_sc_gather(B, T, D, K):
    info = plsc.get_sparse_core_info()
    NC, NS = info.num_cores, info.num_subcores
    NW = NC * NS
    RPW = (B * K) // NW
    WPB = NW // B
    CH = 8
    NCH = RPW // CH

    mesh = plsc.VectorSubcoreMesh(core_axis_name="c", subcore_axis_name="s")

    @functools.partial(
        pl.kernel,
        out_type=(
            jax.ShapeDtypeStruct((B * K, D), jnp.float32),
            jax.ShapeDtypeStruct((B * K,), jnp.int32),
            jax.ShapeDtypeStruct((B * K,), jnp.float32),
        ),
        mesh=mesh,
        compiler_params=pltpu.CompilerParams(needs_layout_passes=False),
        scratch_types=[
            pltpu.VMEM((T,), jnp.int32),
            pltpu.VMEM((T,), jnp.float32),
            pltpu.VMEM((RPW,), jnp.int32),
            pltpu.VMEM((RPW,), jnp.float32),
            pltpu.VMEM((RPW,), jnp.int32),
            pltpu.VMEM((2, CH, D), jnp.float32),
            pltpu.SemaphoreType.DMA,
            pltpu.SemaphoreType.DMA,
            pltpu.SemaphoreType.DMA,
            pltpu.SemaphoreType.DMA,
        ],
    )
    def sc_gather(rank_hbm, g_hbm, hid_hbm, out_hbm, idx_hbm, val_hbm,
                  rank_v, g_v, perm_v, val_v, src_v, rows_v,
                  gsem0, gsem1, osem0, osem1):
        wid = lax.axis_index("s") * NC + lax.axis_index("c")
        b = wid // WPB
        qbase = pl.multiple_of(wid * RPW, RPW)
        lo = qbase - b * K

        pltpu.sync_copy(rank_hbm.at[b], rank_v)
        pltpu.sync_copy(g_hbm.at[b], g_v)

        lane = lax.broadcasted_iota(jnp.int32, (16,), 0)

        def scatter_step(t, carry):
            r = rank_v[pl.ds(pl.multiple_of(t * 16, 16), 16)]
            m = (r >= lo) & (r < lo + RPW)
            plsc.store_scatter(perm_v, [r - lo], lane + t * 16, mask=m)
            return carry

        lax.fori_loop(0, T // 16, scatter_step, 0)

        for t in range(RPW // 16):
            p = perm_v[pl.ds(t * 16, 16)]
            val_v[pl.ds(t * 16, 16)] = plsc.load_gather(g_v, [p])
            src_v[pl.ds(t * 16, 16)] = p + b * T

        pltpu.sync_copy(perm_v, idx_hbm.at[pl.ds(qbase, RPW)])
        pltpu.sync_copy(val_v, val_hbm.at[pl.ds(qbase, RPW)])

        gsems = (gsem0, gsem1)
        osems = (osem0, osem1)

        def start_gather(t):
            return pltpu.async_copy(
                hid_hbm.at[src_v.at[pl.ds(t * CH, CH)]],
                rows_v.at[t % 2], gsems[t % 2])

        def start_write(t):
            return pltpu.async_copy(
                rows_v.at[t % 2],
                out_hbm.at[pl.ds(qbase + t * CH, CH)], osems[t % 2])

        writes = [None, None]
        pending = start_gather(0)
        for t in range(NCH):
            pending.wait()
            writes[t % 2] = start_write(t)
            if t + 1 < NCH:
                if writes[(t + 1) % 2] is not None:
                    writes[(t + 1) % 2].wait()
                    writes[(t + 1) % 2] = None
                pending = start_gather(t + 1)
        for w in writes:
            if w is not None:
                w.wait()

    return sc_gather


def kernel(d_st, d_ch, hidden_states, raw_o_ce, raw_m_cu):
    B, T, D = hidden_states.shape
    K = min(max(1, int(T * _CAPACITY)), T)

    g = _signal(d_st, d_ch, raw_o_ce, raw_m_cu)
    rank = _ranks(g)

    sc = _make_sc_gather(B, T, D, K)
    hid_flat = hidden_states.reshape(B * T, D)
    selected, topk_idx, topk_vals = sc(rank, g, hid_flat)

    batch_idx = jnp.repeat(jnp.arange(B, dtype=jnp.int32), K)
    return selected, batch_idx, topk_idx, topk_vals


# speedup vs baseline: 1.0205x; 1.0205x over previous
import functools

import jax
import jax.numpy as jnp
from jax import lax
from jax.experimental import pallas as pl
from jax.experimental.pallas import tpu as pltpu
from jax.experimental.pallas import tpu_sc as plsc
from jax._src.pallas import mpmd as _mpmd

_BETA_CE = 10.0
_BETA_CU = 10.0
_MA_WINDOW = 100
_CAPACITY = 0.5


def _signal(d_st, d_ch, raw_o_ce, raw_m_cu):
    B, T = d_st.shape
    o_ce_pos = jax.nn.softplus(raw_o_ce)
    m_cu_pos = jax.nn.softplus(raw_m_cu)
    CE = d_st - (d_ch - jnp.log(o_ce_pos + 1e-10))
    W = min(_MA_WINDOW, T)
    if W <= 1:
        ma = d_st
    else:
        pad = jnp.repeat(d_st[:, :1], W - 1, axis=1)
        padded = jnp.concatenate([pad, d_st], axis=1)
        cs = jnp.cumsum(padded, axis=1)
        cs = jnp.concatenate([jnp.zeros((B, 1), dtype=d_st.dtype), cs], axis=1)
        ma = (cs[:, W:] - cs[:, :-W]) / W
    CU = d_st - m_cu_pos * ma
    S_CE = jax.nn.sigmoid(_BETA_CE * CE)
    S_CU = jax.nn.sigmoid(_BETA_CU * CU)
    return S_CE + S_CU - S_CE * S_CU


def _rank_body(g_ref, rank_ref):
    _, T = g_ref.shape
    CH = 256
    jj = lax.broadcasted_iota(jnp.int32, (CH, T), 1)
    g_row = g_ref[0, :].reshape(1, T)
    for c in range(T // CH):
        vi = g_ref[0, c * CH:(c + 1) * CH].reshape(CH, 1)
        ii = lax.broadcasted_iota(jnp.int32, (CH, 1), 0) + (c * CH)
        before = (g_row > vi) | ((g_row == vi) & (jj < ii))
        cnt = jnp.sum(before.astype(jnp.float32), axis=1)
        rank_ref[0, c * CH:(c + 1) * CH] = cnt.astype(jnp.int32)


def _ranks_row(g_row):
    T = g_row.shape[1]
    return pl.pallas_call(
        _rank_body,
        out_shape=jax.ShapeDtypeStruct((1, T), jnp.int32),
    )(g_row)


def _make_sc_batch(b, B, T, D, K, aliased):
    info = plsc.get_sparse_core_info()
    NC, NS = info.num_cores, info.num_subcores
    NW = NC * NS
    RPW = K // NW
    CH = 8
    NCH = RPW // CH
    NB = 3

    mesh = plsc.VectorSubcoreMesh(core_axis_name="c", subcore_axis_name="s")

    def body(rank_hbm, g_hbm, hid_hbm, *rest):
        if aliased:
            (buf_hbm, out_hbm, idx_hbm, val_hbm,
             rank_v, g_v, perm_v, val_v, src_v, rows_v, *sems) = rest
            del buf_hbm
        else:
            (out_hbm, idx_hbm, val_hbm,
             rank_v, g_v, perm_v, val_v, src_v, rows_v, *sems) = rest
        gsems, osems = sems[:NB], sems[NB:]

        wid = lax.axis_index("s") * NC + lax.axis_index("c")
        lo = pl.multiple_of(wid * RPW, RPW)
        qbase = b * K + lo

        pltpu.sync_copy(rank_hbm, rank_v)
        pltpu.sync_copy(g_hbm, g_v)

        lane = lax.broadcasted_iota(jnp.int32, (16,), 0)

        def scatter_step(t, carry):
            r = rank_v[pl.ds(pl.multiple_of(t * 16, 16), 16)]
            m = (r >= lo) & (r < lo + RPW)
            plsc.store_scatter(perm_v, [r - lo], lane + t * 16, mask=m)
            return carry

        lax.fori_loop(0, T // 16, scatter_step, 0)

        for t in range(RPW // 16):
            p = perm_v[pl.ds(t * 16, 16)]
            val_v[pl.ds(t * 16, 16)] = plsc.load_gather(g_v, [p])
            src_v[pl.ds(t * 16, 16)] = p + b * T

        pltpu.sync_copy(perm_v, idx_hbm.at[pl.ds(lo, RPW)])
        pltpu.sync_copy(val_v, val_hbm.at[pl.ds(lo, RPW)])

        def start_gather(t):
            return pltpu.async_copy(
                hid_hbm.at[src_v.at[pl.ds(t * CH, CH)]],
                rows_v.at[t % NB], gsems[t % NB])

        def start_write(t):
            return pltpu.async_copy(
                rows_v.at[t % NB],
                out_hbm.at[pl.ds(qbase + t * CH, CH)], osems[t % NB])

        gd, wd = {}, {}
        for u in range(min(NB - 1, NCH)):
            gd[u] = start_gather(u)
        for t in range(NCH):
            gd[t].wait()
            wd[t] = start_write(t)
            nxt = t + NB - 1
            if nxt < NCH:
                if t >= 1:
                    wd[t - 1].wait()
                    wd[t - 1] = None
                gd[nxt] = start_gather(nxt)
        for t in range(NCH):
            if wd[t] is not None:
                wd[t].wait()

    out_types = (
        jax.ShapeDtypeStruct((B * K, D), jnp.float32),
        jax.ShapeDtypeStruct((K,), jnp.int32),
        jax.ShapeDtypeStruct((K,), jnp.float32),
    )
    scratch = [
        pltpu.VMEM((T,), jnp.int32),
        pltpu.VMEM((T,), jnp.float32),
        pltpu.VMEM((RPW,), jnp.int32),
        pltpu.VMEM((RPW,), jnp.float32),
        pltpu.VMEM((RPW,), jnp.int32),
        pltpu.VMEM((NB, CH, D), jnp.float32),
    ] + [pltpu.SemaphoreType.DMA] * (2 * NB)

    return _mpmd._mpmd_map(
        [(mesh, body)],
        out_types,
        input_output_aliases={3: 0} if aliased else {},
        scratch_types=scratch,
        compiler_params=pltpu.CompilerParams(needs_layout_passes=False),
        name=f"sc_router_b{b}",
    )


def kernel(d_st, d_ch, hidden_states, raw_o_ce, raw_m_cu):
    B, T, D = hidden_states.shape
    K = min(max(1, int(T * _CAPACITY)), T)

    g = _signal(d_st, d_ch, raw_o_ce, raw_m_cu)
    hid_flat = hidden_states.reshape(B * T, D)

    buf = None
    idx_parts, val_parts = [], []
    for b in range(B):
        rank_b = _ranks_row(lax.slice(g, (b, 0), (b + 1, T)))
        sc = _make_sc_batch(b, B, T, D, K, aliased=buf is not None)
        args = (rank_b.reshape(T), g[b].reshape(T), hid_flat)
        if buf is not None:
            args = args + (buf,)
        buf, idx_b, val_b = sc(*args)
        idx_parts.append(idx_b)
        val_parts.append(val_b)

    selected = buf
    topk_idx = jnp.concatenate(idx_parts)
    topk_vals = jnp.concatenate(val_parts)
    batch_idx = jnp.repeat(jnp.arange(B, dtype=jnp.int32), K)
    return selected, batch_idx, topk_idx, topk_vals
